# Initial kernel scaffold; baseline (speedup 1.0000x reference)
import functools

import jax
import jax.numpy as jnp
from jax import lax
from jax.experimental import pallas as pl
from jax.experimental.pallas import tpu as pltpu
from jax.experimental.pallas import tpu_sc as plsc

_PIX = 512 * 512
_NCH = 96
_NWORK = 32
_CH_PER_W = _NCH // _NWORK
_LANES = 16
_NBINS = 256
_CHUNK = 32768
_NCHUNK = _PIX // _CHUNK
_VECS = _CHUNK // _LANES
_ENC = 1 << 19


def _tec_body(img, out, in_buf, out_buf, hist, merged, lut):
    wid = lax.axis_index("s") * 2 + lax.axis_index("c")
    lane = lax.iota(jnp.int32, _LANES)
    lane_off = lane * _NBINS
    ones = jnp.ones((_LANES,), jnp.int32)

    for ci in range(_CH_PER_W):
        base = (wid * _CH_PER_W + ci) * _PIX

        def zero_body(i, _):
            hist[pl.ds(i * _LANES, _LANES)] = jnp.zeros((_LANES,), jnp.int32)
            return _
        lax.fori_loop(0, (_LANES * _NBINS) // _LANES, zero_body, None)

        for j in range(_NCHUNK):
            pltpu.sync_copy(img.at[pl.ds(base + j * _CHUNK, _CHUNK)], in_buf)

            def hist_body(i, _):
                v = in_buf[pl.ds(i * _LANES, _LANES)]
                plsc.addupdate_scatter(hist, [v + lane_off], ones)
                return _
            lax.fori_loop(0, _VECS, hist_body, None, unroll=8)

        enc_max = jnp.int32(0)
        for j2 in range(_NBINS // _LANES):
            acc = jnp.zeros((_LANES,), jnp.int32)
            for l in range(_LANES):
                acc = acc + hist[pl.ds(l * _NBINS + j2 * _LANES, _LANES)]
            merged[pl.ds(j2 * _LANES, _LANES)] = acc
            vi = lane + j2 * _LANES
            enc = jnp.where(acc != 0, vi * _ENC + acc, 0)
            enc_max = jnp.maximum(enc_max, jnp.max(enc))
        last_val = jnp.bitwise_and(enc_max, _ENC - 1)
        step = lax.div(jnp.int32(_PIX) - last_val, jnp.int32(255))
        half = lax.div(step, jnp.int32(2))
        safe_step_v = jnp.broadcast_to(jnp.maximum(step, 1), (_LANES,))
        half_v = jnp.broadcast_to(half, (_LANES,))
        step0_v = jnp.broadcast_to(step == 0, (_LANES,))

        carry = jnp.int32(0)
        for j2 in range(_NBINS // _LANES):
            h = merged[pl.ds(j2 * _LANES, _LANES)]
            c = plsc.cumsum(h) + carry
            carry = jnp.max(c)
            q = lax.div(c - h + half_v, safe_step_v)
            q = jnp.clip(q, 0, 255)
            vi = lane + j2 * _LANES
            lut[pl.ds(j2 * _LANES, _LANES)] = jnp.where(step0_v, vi, q)

        for j in range(_NCHUNK):
            pltpu.sync_copy(img.at[pl.ds(base + j * _CHUNK, _CHUNK)], in_buf)

            def gat_body(i, _):
                v = in_buf[pl.ds(i * _LANES, _LANES)]
                out_buf[pl.ds(i * _LANES, _LANES)] = plsc.load_gather(lut, [v])
                return _
            lax.fori_loop(0, _VECS, gat_body, None, unroll=8)
            pltpu.sync_copy(out_buf, out.at[pl.ds(base + j * _CHUNK, _CHUNK)])


_equalize_sc = functools.partial(
    pl.kernel,
    out_type=jax.ShapeDtypeStruct((_NCH * _PIX,), jnp.int32),
    mesh=plsc.VectorSubcoreMesh(core_axis_name="c", subcore_axis_name="s"),
    scratch_types=[
        pltpu.VMEM((_CHUNK,), jnp.int32),
        pltpu.VMEM((_CHUNK,), jnp.int32),
        pltpu.VMEM((_LANES * _NBINS,), jnp.int32),
        pltpu.VMEM((_NBINS,), jnp.int32),
        pltpu.VMEM((_NBINS,), jnp.int32),
    ],
)(_tec_body)


def kernel(image):
    B, C, H, W = image.shape
    flat = image.reshape(-1)
    out = _equalize_sc(flat)
    return out.reshape(B, C, H, W)


# speedup vs baseline: 271.2734x; 271.2734x over previous
import functools

import jax
import jax.numpy as jnp
from jax import lax
from jax.experimental import pallas as pl
from jax.experimental.pallas import tpu as pltpu
from jax.experimental.pallas import tpu_sc as plsc

_PIX = 512 * 512
_NCH = 96
_NWORK = 32
_CH_PER_W = _NCH // _NWORK
_LANES = 16
_NBINS = 256
_CHUNK = 32768
_NCHUNK = _PIX // _CHUNK
_VECS = _CHUNK // _LANES
_ENC = 1 << 19


def _tec_body(img, out, in_buf, out_buf, hist, merged, lut):
    wid = lax.axis_index("s") * 2 + lax.axis_index("c")
    lane = lax.iota(jnp.int32, _LANES)
    lane_off = lane * _NBINS
    ones = jnp.ones((_LANES,), jnp.int32)

    for ci in range(_CH_PER_W):
        base = (wid * _CH_PER_W + ci) * _PIX

        def zero_body(i, _):
            hist[pl.ds(i * _LANES, _LANES)] = jnp.zeros((_LANES,), jnp.int32)
            return _
        lax.fori_loop(0, (_LANES * _NBINS) // _LANES, zero_body, None)

        for j in range(_NCHUNK):
            pltpu.sync_copy(img.at[pl.ds(base + j * _CHUNK, _CHUNK)], in_buf)

            def hist_body(i, _):
                v = in_buf[pl.ds(i * _LANES, _LANES)]
                plsc.addupdate_scatter(hist, [v + lane_off], ones)
                return _
            lax.fori_loop(0, _VECS, hist_body, None, unroll=8)

        enc_max = jnp.int32(0)
        for j2 in range(_NBINS // _LANES):
            acc = jnp.zeros((_LANES,), jnp.int32)
            for l in range(_LANES):
                acc = acc + hist[pl.ds(l * _NBINS + j2 * _LANES, _LANES)]
            merged[pl.ds(j2 * _LANES, _LANES)] = acc
            vi = lane + j2 * _LANES
            enc = jnp.where(acc != 0, vi * _ENC + acc, 0)
            enc_max = jnp.maximum(enc_max, jnp.max(enc))
        last_val = jnp.bitwise_and(enc_max, _ENC - 1)
        step = lax.div(jnp.int32(_PIX) - last_val, jnp.int32(255))
        half = lax.div(step, jnp.int32(2))
        safe_step_v = jnp.broadcast_to(jnp.maximum(step, 1), (_LANES,))
        half_v = jnp.broadcast_to(half, (_LANES,))
        step0_v = jnp.broadcast_to(step == 0, (_LANES,))

        carry = jnp.int32(0)
        for j2 in range(_NBINS // _LANES):
            h = merged[pl.ds(j2 * _LANES, _LANES)]
            c = plsc.cumsum(h) + carry
            carry = jnp.max(c)
            q = lax.div(c - h + half_v, safe_step_v)
            q = jnp.clip(q, 0, 255)
            vi = lane + j2 * _LANES
            lut[pl.ds(j2 * _LANES, _LANES)] = jnp.where(step0_v, vi, q)

        for j in range(_NCHUNK):
            pltpu.sync_copy(img.at[pl.ds(base + j * _CHUNK, _CHUNK)], in_buf)

            def gat_body(i, _):
                v = in_buf[pl.ds(i * _LANES, _LANES)]
                out_buf[pl.ds(i * _LANES, _LANES)] = plsc.load_gather(lut, [v])
                return _
            lax.fori_loop(0, _VECS, gat_body, None, unroll=8)
            pltpu.sync_copy(out_buf, out.at[pl.ds(base + j * _CHUNK, _CHUNK)])


_equalize_sc = functools.partial(
    pl.kernel,
    out_type=jax.ShapeDtypeStruct((_NCH * _PIX,), jnp.int32),
    mesh=plsc.VectorSubcoreMesh(core_axis_name="c", subcore_axis_name="s"),
    compiler_params=pltpu.CompilerParams(needs_layout_passes=False),
    scratch_types=[
        pltpu.VMEM((_CHUNK,), jnp.int32),
        pltpu.VMEM((_CHUNK,), jnp.int32),
        pltpu.VMEM((_LANES * _NBINS,), jnp.int32),
        pltpu.VMEM((_NBINS,), jnp.int32),
        pltpu.VMEM((_NBINS,), jnp.int32),
    ],
)(_tec_body)


def kernel(image):
    B, C, H, W = image.shape
    flat = image.reshape(-1)
    out = _equalize_sc(flat)
    return out.reshape(B, C, H, W)


# speedup vs baseline: 846.7153x; 3.1213x over previous
import functools

import jax
import jax.numpy as jnp
from jax import lax
from jax.experimental import pallas as pl
from jax.experimental.pallas import tpu as pltpu
from jax.experimental.pallas import tpu_sc as plsc

_PIX = 512 * 512
_NCH = 96
_NWORK = 32
_CH_PER_W = _NCH // _NWORK
_LANES = 16
_NBINS = 256
_CHUNK = 16384
_NCHUNK = _PIX // _CHUNK
_VECS = _CHUNK // _LANES
_ENC = 1 << 19


def _tec_body(img, out, in_a, in_b, out_a, out_b, hist, merged, lut,
              sin0, sin1, sout0, sout1):
    wid = lax.axis_index("s") * 2 + lax.axis_index("c")
    lane = lax.iota(jnp.int32, _LANES)
    lane_off = lane * _NBINS
    ones = jnp.ones((_LANES,), jnp.int32)
    zeros = jnp.zeros((_LANES,), jnp.int32)
    in_bufs = (in_a, in_b)
    out_bufs = (out_a, out_b)
    sins = (sin0, sin1)
    souts = (sout0, sout1)

    def chan_body(ci, _):
        base = (wid * _CH_PER_W + ci) * _PIX

        h_in = [None] * _NCHUNK
        for j in range(2):
            h_in[j] = pltpu.async_copy(
                img.at[pl.ds(base + j * _CHUNK, _CHUNK)], in_bufs[j], sins[j])

        @plsc.parallel_loop(0, _NBINS, unroll=8)
        def _zero(i):
            hist[pl.ds(i * _LANES, _LANES)] = zeros

        for j in range(_NCHUNK):
            b = j % 2
            buf = in_bufs[b]
            h_in[j].wait()

            @plsc.parallel_loop(0, _VECS, unroll=8)
            def _hist(i):
                v = buf[pl.ds(i * _LANES, _LANES)]
                plsc.addupdate_scatter(hist, [v + lane_off], ones)

            if j + 2 < _NCHUNK:
                h_in[j + 2] = pltpu.async_copy(
                    img.at[pl.ds(base + (j + 2) * _CHUNK, _CHUNK)], buf, sins[b])

        g_in = [None] * _NCHUNK
        for j in range(2):
            g_in[j] = pltpu.async_copy(
                img.at[pl.ds(base + j * _CHUNK, _CHUNK)], in_bufs[j], sins[j])

        enc_max = jnp.int32(0)
        for j2 in range(_NBINS // _LANES):
            acc = zeros
            for l in range(_LANES):
                acc = acc + hist[pl.ds(l * _NBINS + j2 * _LANES, _LANES)]
            merged[pl.ds(j2 * _LANES, _LANES)] = acc
            vi = lane + j2 * _LANES
            enc = jnp.where(acc != 0, vi * _ENC + acc, 0)
            enc_max = jnp.maximum(enc_max, jnp.max(enc))
        last_val = jnp.bitwise_and(enc_max, _ENC - 1)
        step = lax.div(jnp.int32(_PIX) - last_val, jnp.int32(255))
        half = lax.div(step, jnp.int32(2))
        safe_step_v = jnp.broadcast_to(jnp.maximum(step, 1), (_LANES,))
        half_v = jnp.broadcast_to(half, (_LANES,))
        step0_v = jnp.broadcast_to(step == 0, (_LANES,))

        carry = jnp.int32(0)
        for j2 in range(_NBINS // _LANES):
            h = merged[pl.ds(j2 * _LANES, _LANES)]
            c = plsc.cumsum(h) + carry
            carry = jnp.max(c)
            q = lax.div(c - h + half_v, safe_step_v)
            q = jnp.clip(q, 0, 255)
            vi = lane + j2 * _LANES
            lut[pl.ds(j2 * _LANES, _LANES)] = jnp.where(step0_v, vi, q)

        g_out = [None] * _NCHUNK
        for j in range(_NCHUNK):
            b = j % 2
            ibuf = in_bufs[b]
            obuf = out_bufs[b]
            g_in[j].wait()
            if j >= 2:
                g_out[j - 2].wait()

            @plsc.parallel_loop(0, _VECS, unroll=8)
            def _gather(i):
                v = ibuf[pl.ds(i * _LANES, _LANES)]
                obuf[pl.ds(i * _LANES, _LANES)] = plsc.load_gather(lut, [v])

            g_out[j] = pltpu.async_copy(
                obuf, out.at[pl.ds(base + j * _CHUNK, _CHUNK)], souts[b])
            if j + 2 < _NCHUNK:
                g_in[j + 2] = pltpu.async_copy(
                    img.at[pl.ds(base + (j + 2) * _CHUNK, _CHUNK)], ibuf, sins[b])
        g_out[_NCHUNK - 2].wait()
        g_out[_NCHUNK - 1].wait()
        return _

    lax.fori_loop(0, _CH_PER_W, chan_body, None)


_equalize_sc = functools.partial(
    pl.kernel,
    out_type=jax.ShapeDtypeStruct((_NCH * _PIX,), jnp.int32),
    mesh=plsc.VectorSubcoreMesh(core_axis_name="c", subcore_axis_name="s"),
    compiler_params=pltpu.CompilerParams(needs_layout_passes=False),
    scratch_types=[
        pltpu.VMEM((_CHUNK,), jnp.int32),
        pltpu.VMEM((_CHUNK,), jnp.int32),
        pltpu.VMEM((_CHUNK,), jnp.int32),
        pltpu.VMEM((_CHUNK,), jnp.int32),
        pltpu.VMEM((_LANES * _NBINS,), jnp.int32),
        pltpu.VMEM((_NBINS,), jnp.int32),
        pltpu.VMEM((_NBINS,), jnp.int32),
        pltpu.SemaphoreType.DMA,
        pltpu.SemaphoreType.DMA,
        pltpu.SemaphoreType.DMA,
        pltpu.SemaphoreType.DMA,
    ],
)(_tec_body)


def kernel(image):
    B, C, H, W = image.shape
    flat = image.reshape(-1)
    out = _equalize_sc(flat)
    return out.reshape(B, C, H, W)


# speedup vs baseline: 1666.9897x; 1.9688x over previous
import functools

import jax
import jax.numpy as jnp
from jax import lax
from jax.experimental import pallas as pl
from jax.experimental.pallas import tpu as pltpu
from jax.experimental.pallas import tpu_sc as plsc

_H = 512
_W = 512
_PIX = _H * _W
_NCH = 96
_NWORK = 32
_CH_PER_W = _NCH // _NWORK
_LANES = 16
_NBINS = 256
_ROWS = 32
_NCHUNK = _H // _ROWS
_VPR = _W // _LANES
_ENC = 1 << 19


def _tec_body(img, out, in_a, in_b, out_a, out_b, hist, merged, lut,
              sin0, sin1, sout0, sout1):
    wid = lax.axis_index("s") * 2 + lax.axis_index("c")
    lane = lax.iota(jnp.int32, _LANES)
    lane_off = lane * _NBINS
    ones = jnp.ones((_LANES,), jnp.int32)
    zeros = jnp.zeros((_LANES,), jnp.int32)
    in_bufs = (in_a, in_b)
    out_bufs = (out_a, out_b)
    sins = (sin0, sin1)
    souts = (sout0, sout1)

    def chan_body(ci, _):
        p = wid * _CH_PER_W + ci

        h_in = [None] * _NCHUNK
        for j in range(2):
            h_in[j] = pltpu.async_copy(
                img.at[p, pl.ds(j * _ROWS, _ROWS)], in_bufs[j], sins[j])

        @plsc.parallel_loop(0, _NBINS, unroll=8)
        def _zero(i):
            hist[pl.ds(i * _LANES, _LANES)] = zeros

        for j in range(_NCHUNK):
            b = j % 2
            buf = in_bufs[b]
            h_in[j].wait()

            @plsc.parallel_loop(0, _ROWS * _VPR, unroll=8)
            def _hist(i):
                r = lax.shift_right_logical(i, 5)
                c = lax.shift_left(jnp.bitwise_and(i, _VPR - 1), 4)
                v = buf[r, pl.ds(c, _LANES)]
                plsc.addupdate_scatter(hist, [v + lane_off], ones)

            if j + 2 < _NCHUNK:
                h_in[j + 2] = pltpu.async_copy(
                    img.at[p, pl.ds((j + 2) * _ROWS, _ROWS)], buf, sins[b])

        g_in = [None] * _NCHUNK
        for j in range(2):
            g_in[j] = pltpu.async_copy(
                img.at[p, pl.ds(j * _ROWS, _ROWS)], in_bufs[j], sins[j])

        enc_max = jnp.int32(0)
        for j2 in range(_NBINS // _LANES):
            acc = zeros
            for l in range(_LANES):
                acc = acc + hist[pl.ds(l * _NBINS + j2 * _LANES, _LANES)]
            merged[pl.ds(j2 * _LANES, _LANES)] = acc
            vi = lane + j2 * _LANES
            enc = jnp.where(acc != 0, vi * _ENC + acc, 0)
            enc_max = jnp.maximum(enc_max, jnp.max(enc))
        last_val = jnp.bitwise_and(enc_max, _ENC - 1)
        step = lax.div(jnp.int32(_PIX) - last_val, jnp.int32(255))
        half = lax.div(step, jnp.int32(2))
        safe_step_v = jnp.broadcast_to(jnp.maximum(step, 1), (_LANES,))
        half_v = jnp.broadcast_to(half, (_LANES,))
        step0_v = jnp.broadcast_to(step == 0, (_LANES,))

        carry = jnp.int32(0)
        for j2 in range(_NBINS // _LANES):
            h = merged[pl.ds(j2 * _LANES, _LANES)]
            c = plsc.cumsum(h) + carry
            carry = jnp.max(c)
            q = lax.div(c - h + half_v, safe_step_v)
            q = jnp.clip(q, 0, 255)
            vi = lane + j2 * _LANES
            lut[pl.ds(j2 * _LANES, _LANES)] = jnp.where(step0_v, vi, q)

        g_out = [None] * _NCHUNK
        for j in range(_NCHUNK):
            b = j % 2
            ibuf = in_bufs[b]
            obuf = out_bufs[b]
            g_in[j].wait()
            if j >= 2:
                g_out[j - 2].wait()

            @plsc.parallel_loop(0, _ROWS * _VPR, unroll=8)
            def _gather(i):
                r = lax.shift_right_logical(i, 5)
                c = lax.shift_left(jnp.bitwise_and(i, _VPR - 1), 4)
                v = ibuf[r, pl.ds(c, _LANES)]
                obuf[r, pl.ds(c, _LANES)] = plsc.load_gather(lut, [v])

            g_out[j] = pltpu.async_copy(
                obuf, out.at[p, pl.ds(j * _ROWS, _ROWS)], souts[b])
            if j + 2 < _NCHUNK:
                g_in[j + 2] = pltpu.async_copy(
                    img.at[p, pl.ds((j + 2) * _ROWS, _ROWS)], ibuf, sins[b])
        g_out[_NCHUNK - 2].wait()
        g_out[_NCHUNK - 1].wait()
        return _

    lax.fori_loop(0, _CH_PER_W, chan_body, None)


_equalize_sc = functools.partial(
    pl.kernel,
    out_type=jax.ShapeDtypeStruct((_NCH, _H, _W), jnp.int32),
    mesh=plsc.VectorSubcoreMesh(core_axis_name="c", subcore_axis_name="s"),
    compiler_params=pltpu.CompilerParams(needs_layout_passes=False),
    scratch_types=[
        pltpu.VMEM((_ROWS, _W), jnp.int32),
        pltpu.VMEM((_ROWS, _W), jnp.int32),
        pltpu.VMEM((_ROWS, _W), jnp.int32),
        pltpu.VMEM((_ROWS, _W), jnp.int32),
        pltpu.VMEM((_LANES * _NBINS,), jnp.int32),
        pltpu.VMEM((_NBINS,), jnp.int32),
        pltpu.VMEM((_NBINS,), jnp.int32),
        pltpu.SemaphoreType.DMA,
        pltpu.SemaphoreType.DMA,
        pltpu.SemaphoreType.DMA,
        pltpu.SemaphoreType.DMA,
    ],
)(_tec_body)


def kernel(image):
    B, C, H, W = image.shape
    planes = image.reshape(B * C, H, W)
    out = _equalize_sc(planes)
    return out.reshape(B, C, H, W)


# speedup vs baseline: 1888.8327x; 1.1331x over previous
import functools

import jax
import jax.numpy as jnp
from jax import lax
from jax.experimental import pallas as pl
from jax.experimental.pallas import tpu as pltpu
from jax.experimental.pallas import tpu_sc as plsc

_H = 512
_W = 512
_PIX = _H * _W
_NCH = 96
_CH_PER_W = 3
_LANES = 16
_NBINS = 256
_ROWS_IN = 32
_NCHUNK_IN = _H // _ROWS_IN
_GRP_IN = _ROWS_IN * (_W // _LANES) // 4
_ROWS_OUT = 16
_NCHUNK_OUT = _H // _ROWS_OUT
_GRP_OUT = _ROWS_OUT * (_W // _LANES) // 4
_ENC = 1 << 19


def _tec_body(img, out, in_a, in_b, out_a, out_b, packed, hist, merged, lut,
              sin0, sin1, sout0, sout1):
    wid = lax.axis_index("s") * 2 + lax.axis_index("c")
    lane = lax.iota(jnp.int32, _LANES)
    lane_off = lane * _NBINS
    ones = jnp.ones((_LANES,), jnp.int32)
    zeros = jnp.zeros((_LANES,), jnp.int32)
    in_bufs = (in_a, in_b)
    out_bufs = (out_a, out_b)
    sins = (sin0, sin1)
    souts = (sout0, sout1)

    def chan_body(ci, _):
        p = wid * _CH_PER_W + ci

        h_in = [None] * _NCHUNK_IN
        for j in range(2):
            h_in[j] = pltpu.async_copy(
                img.at[p, pl.ds(j * _ROWS_IN, _ROWS_IN)], in_bufs[j], sins[j])

        @plsc.parallel_loop(0, _NBINS, unroll=8)
        def _zero(i):
            hist[pl.ds(i * _LANES, _LANES)] = zeros

        for j in range(_NCHUNK_IN):
            b = j % 2
            buf = in_bufs[b]
            h_in[j].wait()

            @plsc.parallel_loop(0, _GRP_IN, unroll=2)
            def _hist(g):
                r = lax.shift_right_logical(g, 3)
                c0 = lax.shift_left(jnp.bitwise_and(g, 7), 6)
                vs = []
                for k in range(4):
                    v = buf[r, pl.ds(c0 + k * _LANES, _LANES)]
                    plsc.addupdate_scatter(hist, [v + lane_off], ones)
                    vs.append(v)
                w = (vs[0] | lax.shift_left(vs[1], 8)
                     | lax.shift_left(vs[2], 16) | lax.shift_left(vs[3], 24))
                packed[pl.ds(j * (_GRP_IN * _LANES) + g * _LANES, _LANES)] = w

            if j + 2 < _NCHUNK_IN:
                h_in[j + 2] = pltpu.async_copy(
                    img.at[p, pl.ds((j + 2) * _ROWS_IN, _ROWS_IN)], buf,
                    sins[b])

        enc_max = jnp.int32(0)
        for j2 in range(_NBINS // _LANES):
            acc = zeros
            for l in range(_LANES):
                acc = acc + hist[pl.ds(l * _NBINS + j2 * _LANES, _LANES)]
            merged[pl.ds(j2 * _LANES, _LANES)] = acc
            vi = lane + j2 * _LANES
            enc = jnp.where(acc != 0, vi * _ENC + acc, 0)
            enc_max = jnp.maximum(enc_max, jnp.max(enc))
        last_val = jnp.bitwise_and(enc_max, _ENC - 1)
        step = lax.div(jnp.int32(_PIX) - last_val, jnp.int32(255))
        half = lax.div(step, jnp.int32(2))
        safe_step_v = jnp.broadcast_to(jnp.maximum(step, 1), (_LANES,))
        half_v = jnp.broadcast_to(half, (_LANES,))
        step0_v = jnp.broadcast_to(step == 0, (_LANES,))

        carry = jnp.int32(0)
        for j2 in range(_NBINS // _LANES):
            h = merged[pl.ds(j2 * _LANES, _LANES)]
            c = plsc.cumsum(h) + carry
            carry = jnp.max(c)
            q = lax.div(c - h + half_v, safe_step_v)
            q = jnp.clip(q, 0, 255)
            vi = lane + j2 * _LANES
            lut[pl.ds(j2 * _LANES, _LANES)] = jnp.where(step0_v, vi, q)

        g_out = [None] * _NCHUNK_OUT
        for j in range(_NCHUNK_OUT):
            b = j % 2
            obuf = out_bufs[b]
            if j >= 2:
                g_out[j - 2].wait()

            @plsc.parallel_loop(0, _GRP_OUT, unroll=2)
            def _gather(g):
                r = lax.shift_right_logical(g, 3)
                c0 = lax.shift_left(jnp.bitwise_and(g, 7), 6)
                w = packed[pl.ds(j * (_GRP_OUT * _LANES) + g * _LANES, _LANES)]
                v0 = jnp.bitwise_and(w, 255)
                v1 = jnp.bitwise_and(lax.shift_right_logical(w, 8), 255)
                v2 = jnp.bitwise_and(lax.shift_right_logical(w, 16), 255)
                v3 = lax.shift_right_logical(w, 24)
                for k, v in enumerate((v0, v1, v2, v3)):
                    obuf[r, pl.ds(c0 + k * _LANES, _LANES)] = (
                        plsc.load_gather(lut, [v]))

            g_out[j] = pltpu.async_copy(
                obuf, out.at[p, pl.ds(j * _ROWS_OUT, _ROWS_OUT)], souts[b])
        g_out[_NCHUNK_OUT - 2].wait()
        g_out[_NCHUNK_OUT - 1].wait()
        return _

    lax.fori_loop(0, _CH_PER_W, chan_body, None)


_equalize_sc = functools.partial(
    pl.kernel,
    out_type=jax.ShapeDtypeStruct((_NCH, _H, _W), jnp.int32),
    mesh=plsc.VectorSubcoreMesh(core_axis_name="c", subcore_axis_name="s"),
    compiler_params=pltpu.CompilerParams(needs_layout_passes=False),
    scratch_types=[
        pltpu.VMEM((_ROWS_IN, _W), jnp.int32),
        pltpu.VMEM((_ROWS_IN, _W), jnp.int32),
        pltpu.VMEM((_ROWS_OUT, _W), jnp.int32),
        pltpu.VMEM((_ROWS_OUT, _W), jnp.int32),
        pltpu.VMEM((_PIX // 4, ), jnp.int32),
        pltpu.VMEM((_LANES * _NBINS,), jnp.int32),
        pltpu.VMEM((_NBINS,), jnp.int32),
        pltpu.VMEM((_NBINS,), jnp.int32),
        pltpu.SemaphoreType.DMA,
        pltpu.SemaphoreType.DMA,
        pltpu.SemaphoreType.DMA,
        pltpu.SemaphoreType.DMA,
    ],
)(_tec_body)


def kernel(image):
    B, C, H, W = image.shape
    planes = image.reshape(B * C, H, W)
    out = _equalize_sc(planes)
    return out.reshape(B, C, H, W)


# speedup vs baseline: 1890.5890x; 1.0009x over previous
import functools

import jax
import jax.numpy as jnp
from jax import lax
from jax.experimental import pallas as pl
from jax.experimental.pallas import tpu as pltpu
from jax.experimental.pallas import tpu_sc as plsc

_H = 512
_W = 512
_PIX = _H * _W
_NCH = 96
_CH_PER_W = 3
_LANES = 16
_NBINS = 256
_ROWS_IN = 32
_NCHUNK_IN = _H // _ROWS_IN
_GRP_IN = _ROWS_IN * (_W // _LANES) // 4
_ROWS_OUT = 16
_NCHUNK_OUT = _H // _ROWS_OUT
_GRP_OUT = _ROWS_OUT * (_W // _LANES) // 4
_ENC = 1 << 19


def _tec_body(img, out, in_a, in_b, out_a, out_b, out_c, packed, hist,
              merged, lut, sin0, sin1, sout0, sout1, sout2):
    wid = lax.axis_index("s") * 2 + lax.axis_index("c")
    lane = lax.iota(jnp.int32, _LANES)
    lane_off = lane * _NBINS
    ones = jnp.ones((_LANES,), jnp.int32)
    zeros = jnp.zeros((_LANES,), jnp.int32)
    in_bufs = (in_a, in_b)
    out_bufs = (out_a, out_b, out_c)
    sins = (sin0, sin1)
    souts = (sout0, sout1, sout2)

    def chan_body(ci, _):
        p = wid * _CH_PER_W + ci

        h_in = [None] * _NCHUNK_IN
        for j in range(2):
            h_in[j] = pltpu.async_copy(
                img.at[p, pl.ds(j * _ROWS_IN, _ROWS_IN)], in_bufs[j], sins[j])

        @plsc.parallel_loop(0, _NBINS, unroll=8)
        def _zero(i):
            hist[pl.ds(i * _LANES, _LANES)] = zeros

        for j in range(_NCHUNK_IN):
            b = j % 2
            buf = in_bufs[b]
            h_in[j].wait()

            @plsc.parallel_loop(0, _GRP_IN, unroll=2)
            def _hist(g):
                r = lax.shift_right_logical(g, 3)
                c0 = lax.shift_left(jnp.bitwise_and(g, 7), 6)
                vs = []
                for k in range(4):
                    v = buf[r, pl.ds(c0 + k * _LANES, _LANES)]
                    plsc.addupdate_scatter(hist, [v + lane_off], ones)
                    vs.append(v)
                w = (vs[0] | lax.shift_left(vs[1], 8)
                     | lax.shift_left(vs[2], 16) | lax.shift_left(vs[3], 24))
                packed[pl.ds(j * (_GRP_IN * _LANES) + g * _LANES, _LANES)] = w

            if j + 2 < _NCHUNK_IN:
                h_in[j + 2] = pltpu.async_copy(
                    img.at[p, pl.ds((j + 2) * _ROWS_IN, _ROWS_IN)], buf,
                    sins[b])

        enc_max = jnp.int32(0)
        for j2 in range(_NBINS // _LANES):
            acc = zeros
            for l in range(_LANES):
                acc = acc + hist[pl.ds(l * _NBINS + j2 * _LANES, _LANES)]
            merged[pl.ds(j2 * _LANES, _LANES)] = acc
            vi = lane + j2 * _LANES
            enc = jnp.where(acc != 0, vi * _ENC + acc, 0)
            enc_max = jnp.maximum(enc_max, jnp.max(enc))
        last_val = jnp.bitwise_and(enc_max, _ENC - 1)
        step = lax.div(jnp.int32(_PIX) - last_val, jnp.int32(255))
        half = lax.div(step, jnp.int32(2))
        safe_step_v = jnp.broadcast_to(jnp.maximum(step, 1), (_LANES,))
        half_v = jnp.broadcast_to(half, (_LANES,))
        step0_v = jnp.broadcast_to(step == 0, (_LANES,))

        carry = jnp.int32(0)
        for j2 in range(_NBINS // _LANES):
            h = merged[pl.ds(j2 * _LANES, _LANES)]
            c = plsc.cumsum(h) + carry
            carry = jnp.max(c)
            q = lax.div(c - h + half_v, safe_step_v)
            q = jnp.clip(q, 0, 255)
            vi = lane + j2 * _LANES
            lut[pl.ds(j2 * _LANES, _LANES)] = jnp.where(step0_v, vi, q)

        g_out = [None] * _NCHUNK_OUT
        for j in range(_NCHUNK_OUT):
            b = j % 3
            obuf = out_bufs[b]
            if j >= 3:
                g_out[j - 3].wait()

            @plsc.parallel_loop(0, _GRP_OUT, unroll=2)
            def _gather(g):
                r = lax.shift_right_logical(g, 3)
                c0 = lax.shift_left(jnp.bitwise_and(g, 7), 6)
                w = packed[pl.ds(j * (_GRP_OUT * _LANES) + g * _LANES, _LANES)]
                v0 = jnp.bitwise_and(w, 255)
                v1 = jnp.bitwise_and(lax.shift_right_logical(w, 8), 255)
                v2 = jnp.bitwise_and(lax.shift_right_logical(w, 16), 255)
                v3 = lax.shift_right_logical(w, 24)
                for k, v in enumerate((v0, v1, v2, v3)):
                    obuf[r, pl.ds(c0 + k * _LANES, _LANES)] = (
                        plsc.load_gather(lut, [v]))

            g_out[j] = pltpu.async_copy(
                obuf, out.at[p, pl.ds(j * _ROWS_OUT, _ROWS_OUT)], souts[b])
        g_out[_NCHUNK_OUT - 3].wait()
        g_out[_NCHUNK_OUT - 2].wait()
        g_out[_NCHUNK_OUT - 1].wait()
        return _

    lax.fori_loop(0, _CH_PER_W, chan_body, None)


_equalize_sc = functools.partial(
    pl.kernel,
    out_type=jax.ShapeDtypeStruct((_NCH, _H, _W), jnp.int32),
    mesh=plsc.VectorSubcoreMesh(core_axis_name="c", subcore_axis_name="s"),
    compiler_params=pltpu.CompilerParams(needs_layout_passes=False),
    scratch_types=[
        pltpu.VMEM((_ROWS_IN, _W), jnp.int32),
        pltpu.VMEM((_ROWS_IN, _W), jnp.int32),
        pltpu.VMEM((_ROWS_OUT, _W), jnp.int32),
        pltpu.VMEM((_ROWS_OUT, _W), jnp.int32),
        pltpu.VMEM((_ROWS_OUT, _W), jnp.int32),
        pltpu.VMEM((_PIX // 4, ), jnp.int32),
        pltpu.VMEM((_LANES * _NBINS,), jnp.int32),
        pltpu.VMEM((_NBINS,), jnp.int32),
        pltpu.VMEM((_NBINS,), jnp.int32),
        pltpu.SemaphoreType.DMA,
        pltpu.SemaphoreType.DMA,
        pltpu.SemaphoreType.DMA,
        pltpu.SemaphoreType.DMA,
        pltpu.SemaphoreType.DMA,
    ],
)(_tec_body)


def kernel(image):
    B, C, H, W = image.shape
    planes = image.reshape(B * C, H, W)
    out = _equalize_sc(planes)
    return out.reshape(B, C, H, W)


# speedup vs baseline: 2138.6382x; 1.1312x over previous
import functools

import jax
import jax.numpy as jnp
from jax import lax
from jax.experimental import pallas as pl
from jax.experimental.pallas import tpu as pltpu
from jax.experimental.pallas import tpu_sc as plsc

_H = 512
_W = 512
_PIX = _H * _W
_NCH = 96
_CH_PER_W = 3
_LANES = 16
_NBINS = 256
_ROWS_IN = 32
_NCHUNK_IN = _H // _ROWS_IN
_GRP_IN = _ROWS_IN * (_W // _LANES) // 4
_ROWS_OUT = 16
_NCHUNK_OUT = _H // _ROWS_OUT
_GRP_OUT = _ROWS_OUT * (_W // _LANES) // 4
_ENC = 1 << 19


def _tec_body(img, out, in_a, in_b, out_a, out_b, packed, hist,
              merged, lut16, sin0, sin1, sout0, sout1):
    wid = lax.axis_index("s") * 2 + lax.axis_index("c")
    lane = lax.iota(jnp.int32, _LANES)
    lane_off = lane * _NBINS
    ones = jnp.ones((_LANES,), jnp.int32)
    zeros = jnp.zeros((_LANES,), jnp.int32)
    in_bufs = (in_a, in_b)
    out_bufs = (out_a, out_b)
    sins = (sin0, sin1)
    souts = (sout0, sout1)

    def chan_body(ci, _):
        p = wid * _CH_PER_W + ci

        h_in = [None] * _NCHUNK_IN
        for j in range(2):
            h_in[j] = pltpu.async_copy(
                img.at[p, pl.ds(j * _ROWS_IN, _ROWS_IN)], in_bufs[j], sins[j])

        @plsc.parallel_loop(0, _NBINS, unroll=8)
        def _zero(i):
            hist[pl.ds(i * _LANES, _LANES)] = zeros

        for j in range(_NCHUNK_IN):
            b = j % 2
            buf = in_bufs[b]
            h_in[j].wait()

            @plsc.parallel_loop(0, _GRP_IN, unroll=2)
            def _hist(g):
                r = lax.shift_right_logical(g, 3)
                c0 = lax.shift_left(jnp.bitwise_and(g, 7), 6)
                vs = []
                for k in range(4):
                    v = buf[r, pl.ds(c0 + k * _LANES, _LANES)]
                    plsc.addupdate_scatter(
                        hist, [lax.shift_left(v, 4) + lane], ones)
                    vs.append(v)
                w = (vs[0] | lax.shift_left(vs[1], 8)
                     | lax.shift_left(vs[2], 16) | lax.shift_left(vs[3], 24))
                packed[pl.ds(j * (_GRP_IN * _LANES) + g * _LANES, _LANES)] = w

            if j + 2 < _NCHUNK_IN:
                h_in[j + 2] = pltpu.async_copy(
                    img.at[p, pl.ds((j + 2) * _ROWS_IN, _ROWS_IN)], buf,
                    sins[b])

        @plsc.parallel_loop(0, _NBINS, unroll=4)
        def _merge(v):
            hv = hist[pl.ds(lax.shift_left(v, 4), _LANES)]
            merged[v] = jnp.sum(hv)

        def _enc_body(v, m):
            h = merged[v]
            enc = jnp.where(h != 0, v * _ENC + h, 0)
            return jnp.maximum(m, enc)
        enc_max = lax.fori_loop(0, _NBINS, _enc_body, jnp.int32(0))
        last_val = jnp.bitwise_and(enc_max, _ENC - 1)
        step = lax.div(jnp.int32(_PIX) - last_val, jnp.int32(255))
        half = lax.div(step, jnp.int32(2))
        safe_step = jnp.maximum(step, 1)
        step_is0 = step == 0

        def _lut_body(v, cum):
            h = merged[v]
            q = lax.div(cum + half, safe_step)
            q = jnp.clip(q, 0, 255)
            s = jnp.where(step_is0, v, q)
            lut16[pl.ds(lax.shift_left(v, 4), _LANES)] = jnp.broadcast_to(
                s, (_LANES,))
            return cum + h
        lax.fori_loop(0, _NBINS, _lut_body, jnp.int32(0))

        g_out = [None] * _NCHUNK_OUT
        for j in range(_NCHUNK_OUT):
            b = j % 2
            obuf = out_bufs[b]
            if j >= 2:
                g_out[j - 2].wait()

            @plsc.parallel_loop(0, _GRP_OUT, unroll=2)
            def _gather(g):
                r = lax.shift_right_logical(g, 3)
                c0 = lax.shift_left(jnp.bitwise_and(g, 7), 6)
                w = packed[pl.ds(j * (_GRP_OUT * _LANES) + g * _LANES, _LANES)]
                i0 = jnp.bitwise_and(lax.shift_left(w, 4), 0xFF0)
                i1 = jnp.bitwise_and(lax.shift_right_logical(w, 4), 0xFF0)
                i2 = jnp.bitwise_and(lax.shift_right_logical(w, 12), 0xFF0)
                i3 = jnp.bitwise_and(lax.shift_right_logical(w, 20), 0xFF0)
                for k, iv in enumerate((i0, i1, i2, i3)):
                    obuf[r, pl.ds(c0 + k * _LANES, _LANES)] = (
                        plsc.load_gather(lut16, [iv + lane]))

            g_out[j] = pltpu.async_copy(
                obuf, out.at[p, pl.ds(j * _ROWS_OUT, _ROWS_OUT)], souts[b])
        g_out[_NCHUNK_OUT - 2].wait()
        g_out[_NCHUNK_OUT - 1].wait()
        return _

    lax.fori_loop(0, _CH_PER_W, chan_body, None)


_equalize_sc = functools.partial(
    pl.kernel,
    out_type=jax.ShapeDtypeStruct((_NCH, _H, _W), jnp.int32),
    mesh=plsc.VectorSubcoreMesh(core_axis_name="c", subcore_axis_name="s"),
    compiler_params=pltpu.CompilerParams(needs_layout_passes=False),
    scratch_types=[
        pltpu.VMEM((_ROWS_IN, _W), jnp.int32),
        pltpu.VMEM((_ROWS_IN, _W), jnp.int32),
        pltpu.VMEM((_ROWS_OUT, _W), jnp.int32),
        pltpu.VMEM((_ROWS_OUT, _W), jnp.int32),
        pltpu.VMEM((_PIX // 4, ), jnp.int32),
        pltpu.VMEM((_LANES * _NBINS,), jnp.int32),
        pltpu.SMEM((_NBINS,), jnp.int32),
        pltpu.VMEM((_LANES * _NBINS,), jnp.int32),
        pltpu.SemaphoreType.DMA,
        pltpu.SemaphoreType.DMA,
        pltpu.SemaphoreType.DMA,
        pltpu.SemaphoreType.DMA,
    ],
)(_tec_body)


def kernel(image):
    B, C, H, W = image.shape
    planes = image.reshape(B * C, H, W)
    out = _equalize_sc(planes)
    return out.reshape(B, C, H, W)
